# Initial kernel scaffold; baseline (speedup 1.0000x reference)
import jax
import jax.numpy as jnp
from jax.experimental import pallas as pl


def kernel(role, race, gend, align, role_table, race_table, gend_table, align_table):
    raise NotImplementedError("write your pallas kernel here")


# speedup vs baseline: 7.0823x; 7.0823x over previous
import functools

import jax
import jax.numpy as jnp
from jax import lax
from jax.experimental import pallas as pl
from jax.experimental.pallas import tpu as pltpu, tpu_sc as plsc

ROLE_CAD, ROLE_EMB = 13, 8
RACE_CAD, RACE_EMB = 5, 4
GEND_CAD, GEND_EMB = 3, 2
ALIGN_CAD, ALIGN_EMB = 3, 2
B = 16384
D = ROLE_EMB + RACE_EMB + GEND_EMB + ALIGN_EMB

NC, NS, L = 2, 16, 16
NW = NC * NS
BPW = B // NW
CHUNKS = BPW // L


def _hero_body(role_h, race_h, gend_h, align_h, rt_h, ct_h, gt_h, at_h,
               out_h, rt_v, ct_v, gt_v, at_v, ri_v, ci_v, gi_v, ai_v,
               out_v):
    wid = lax.axis_index("s") * NC + lax.axis_index("c")
    base = wid * BPW

    pltpu.sync_copy(rt_h, rt_v)
    pltpu.sync_copy(ct_h, ct_v)
    pltpu.sync_copy(gt_h, gt_v)
    pltpu.sync_copy(at_h, at_v)
    pltpu.sync_copy(role_h.at[pl.ds(base, BPW)], ri_v)
    pltpu.sync_copy(race_h.at[pl.ds(base, BPW)], ci_v)
    pltpu.sync_copy(gend_h.at[pl.ds(base, BPW)], gi_v)
    pltpu.sync_copy(align_h.at[pl.ds(base, BPW)], ai_v)

    lane = lax.iota(jnp.int32, L)

    def chunk_body(k, carry):
        e0 = k * L
        rows = e0 + lane
        r = ri_v[pl.ds(e0, L)]
        c = ci_v[pl.ds(e0, L)]
        g = gi_v[pl.ds(e0, L)]
        a = ai_v[pl.ds(e0, L)]
        a = jnp.minimum(jnp.maximum(a + 1, 0), ALIGN_CAD - 1)
        for col in range(D):
            col_vec = jnp.full((L,), col, jnp.int32)
            if col < ROLE_EMB:
                vals = plsc.load_gather(rt_v, [r * ROLE_EMB + col])
            elif col < ROLE_EMB + RACE_EMB:
                vals = plsc.load_gather(ct_v, [c * RACE_EMB + (col - ROLE_EMB)])
            elif col < ROLE_EMB + RACE_EMB + GEND_EMB:
                vals = plsc.load_gather(
                    gt_v, [g * GEND_EMB + (col - ROLE_EMB - RACE_EMB)])
            else:
                vals = plsc.load_gather(
                    at_v,
                    [a * ALIGN_EMB + (col - ROLE_EMB - RACE_EMB - GEND_EMB)])
            plsc.store_scatter(out_v, [rows * D + col_vec], vals)
        return carry

    lax.fori_loop(0, CHUNKS, chunk_body, 0)

    pltpu.sync_copy(out_v, out_h.at[pl.ds(base * D, BPW * D)])


_hero = functools.partial(
    pl.kernel,
    out_type=jax.ShapeDtypeStruct((B * D,), jnp.float32),
    mesh=plsc.VectorSubcoreMesh(core_axis_name="c", subcore_axis_name="s"),
    compiler_params=pltpu.CompilerParams(needs_layout_passes=False),
    scratch_types=[
        pltpu.VMEM((ROLE_CAD * ROLE_EMB,), jnp.float32),
        pltpu.VMEM((RACE_CAD * RACE_EMB,), jnp.float32),
        pltpu.VMEM((GEND_CAD * GEND_EMB,), jnp.float32),
        pltpu.VMEM((ALIGN_CAD * ALIGN_EMB,), jnp.float32),
        pltpu.VMEM((BPW,), jnp.int32),
        pltpu.VMEM((BPW,), jnp.int32),
        pltpu.VMEM((BPW,), jnp.int32),
        pltpu.VMEM((BPW,), jnp.int32),
        pltpu.VMEM((BPW * D,), jnp.float32),
    ],
)(_hero_body)


def kernel(role, race, gend, align, role_table, race_table, gend_table,
           align_table):
    out_flat = _hero(role.astype(jnp.int32), race.astype(jnp.int32),
                     gend.astype(jnp.int32), align.astype(jnp.int32),
                     role_table.reshape(-1), race_table.reshape(-1),
                     gend_table.reshape(-1), align_table.reshape(-1))
    return out_flat.reshape(B, D)


# speedup vs baseline: 8.4592x; 1.1944x over previous
import functools

import jax
import jax.numpy as jnp
from jax import lax
from jax.experimental import pallas as pl
from jax.experimental.pallas import tpu as pltpu, tpu_sc as plsc

ROLE_CAD, ROLE_EMB = 13, 8
RACE_CAD, RACE_EMB = 5, 4
GEND_CAD, GEND_EMB = 3, 2
ALIGN_CAD, ALIGN_EMB = 3, 2
B = 16384
D = ROLE_EMB + RACE_EMB + GEND_EMB + ALIGN_EMB

NC, NS, L = 2, 16, 16
NW = NC * NS
BPW = B // NW
CHUNKS = BPW // L

FUSED = ROLE_CAD * RACE_CAD * GEND_CAD * ALIGN_CAD
ROWS_PER_SUB = (FUSED + NS - 1) // NS
FUSED_PAD = ROWS_PER_SUB * NS

OFF_ROLE, OFF_RACE, OFF_GEND, OFF_ALIGN = 0, 104, 128, 136
FLAT_LEN = 144

STREAMS = 4
ROWS_PER_STREAM = BPW // STREAMS


def _hero_body(role_h, race_h, gend_h, align_h, rt_h, ct_h, gt_h, at_h,
               out_h, flat_v, stage_v, ri_v, ci_v, gi_v, ai_v, fidx_v,
               out_v, shared_v, sem):
    cid = lax.axis_index("c")
    sid = lax.axis_index("s")
    wid = sid * NC + cid
    base = wid * BPW

    pltpu.sync_copy(rt_h, flat_v.at[pl.ds(OFF_ROLE, ROLE_CAD * ROLE_EMB)])
    pltpu.sync_copy(ct_h, flat_v.at[pl.ds(OFF_RACE, RACE_CAD * RACE_EMB)])
    pltpu.sync_copy(gt_h, flat_v.at[pl.ds(OFF_GEND, GEND_CAD * GEND_EMB)])
    pltpu.sync_copy(at_h, flat_v.at[pl.ds(OFF_ALIGN, ALIGN_CAD * ALIGN_EMB)])

    ri_d = pltpu.async_copy(role_h.at[pl.ds(base, BPW)], ri_v, sem)
    ci_d = pltpu.async_copy(race_h.at[pl.ds(base, BPW)], ci_v, sem)
    gi_d = pltpu.async_copy(gend_h.at[pl.ds(base, BPW)], gi_v, sem)
    ai_d = pltpu.async_copy(align_h.at[pl.ds(base, BPW)], ai_v, sem)

    lane = lax.iota(jnp.int32, L)
    m_role = lane < ROLE_EMB
    m_race = lane < ROLE_EMB + RACE_EMB
    m_gend = lane < ROLE_EMB + RACE_EMB + GEND_EMB
    l_role = lane
    l_race = lane - ROLE_EMB
    l_gend = lane - (ROLE_EMB + RACE_EMB)
    l_align = lane - (ROLE_EMB + RACE_EMB + GEND_EMB)
    f0 = sid * ROWS_PER_SUB
    for j in range(ROWS_PER_SUB):
        f = jnp.minimum(f0 + j, FUSED - 1)
        r = f // (RACE_CAD * GEND_CAD * ALIGN_CAD)
        rem = f % (RACE_CAD * GEND_CAD * ALIGN_CAD)
        c = rem // (GEND_CAD * ALIGN_CAD)
        rem = rem % (GEND_CAD * ALIGN_CAD)
        g = rem // ALIGN_CAD
        a = rem % ALIGN_CAD
        offs = jnp.where(
            m_role, OFF_ROLE + r * ROLE_EMB + l_role,
            jnp.where(
                m_race, OFF_RACE + c * RACE_EMB + l_race,
                jnp.where(m_gend, OFF_GEND + g * GEND_EMB + l_gend,
                          OFF_ALIGN + a * ALIGN_EMB + l_align)))
        stage_v[j, :] = plsc.load_gather(flat_v, [offs])
    pltpu.sync_copy(stage_v, shared_v.at[pl.ds(f0, ROWS_PER_SUB)])

    ri_d.wait()
    ci_d.wait()
    gi_d.wait()
    ai_d.wait()
    for k in range(CHUNKS):
        e0 = k * L
        r = ri_v[pl.ds(e0, L)]
        c = ci_v[pl.ds(e0, L)]
        g = gi_v[pl.ds(e0, L)]
        a = ai_v[pl.ds(e0, L)]
        a = jnp.minimum(jnp.maximum(a + 1, 0), ALIGN_CAD - 1)
        f = ((r * RACE_CAD + c) * GEND_CAD + g) * ALIGN_CAD + a
        fidx_v[k // 8, pl.ds((k % 8) * L, L)] = f

    plsc.subcore_barrier()

    descs = []
    for j in range(STREAMS):
        descs.append(
            pltpu.async_copy(
                shared_v.at[fidx_v.at[j]],
                out_v.at[pl.ds(j * ROWS_PER_STREAM, ROWS_PER_STREAM)], sem))
    for d in descs:
        d.wait()

    pltpu.sync_copy(out_v, out_h.at[pl.ds(base, BPW)])


_hero = functools.partial(
    pl.kernel,
    out_type=jax.ShapeDtypeStruct((B, D), jnp.float32),
    mesh=plsc.VectorSubcoreMesh(core_axis_name="c", subcore_axis_name="s"),
    compiler_params=pltpu.CompilerParams(needs_layout_passes=False),
    scratch_types=[
        pltpu.VMEM((FLAT_LEN,), jnp.float32),
        pltpu.VMEM((ROWS_PER_SUB, D), jnp.float32),
        pltpu.VMEM((BPW,), jnp.int32),
        pltpu.VMEM((BPW,), jnp.int32),
        pltpu.VMEM((BPW,), jnp.int32),
        pltpu.VMEM((BPW,), jnp.int32),
        pltpu.VMEM((STREAMS, ROWS_PER_STREAM), jnp.int32),
        pltpu.VMEM((BPW, D), jnp.float32),
        pltpu.VMEM_SHARED((FUSED_PAD, D), jnp.float32),
        pltpu.SemaphoreType.DMA,
    ],
)(_hero_body)


def kernel(role, race, gend, align, role_table, race_table, gend_table,
           align_table):
    return _hero(role.astype(jnp.int32), race.astype(jnp.int32),
                 gend.astype(jnp.int32), align.astype(jnp.int32),
                 role_table.reshape(-1), race_table.reshape(-1),
                 gend_table.reshape(-1), align_table.reshape(-1))
